# Initial kernel scaffold; baseline (speedup 1.0000x reference)
import functools

import jax
import jax.numpy as jnp
from jax import lax
from jax.experimental import pallas as pl
from jax.experimental.pallas import tpu as pltpu
from jax.experimental.pallas import tpu_sc as plsc

N = 10000
E = 320000
D = 128
G = 16

NC = 2
NS = 16
EPW = E // (NC * NS)
W = 80
NCHUNK = EPW // W
RPT = N // NS

BLK = 400
GRID = N // BLK


def _vmesh():
    return plsc.VectorSubcoreMesh(core_axis_name="c", subcore_axis_name="s")


def _sc_degree(dst, zeros_n, ones_w):
    @functools.partial(
        pl.kernel,
        out_type=jax.ShapeDtypeStruct((NC, N), jnp.float32),
        mesh=_vmesh(),
        scratch_types=[
            pltpu.VMEM((W,), jnp.int32),
            pltpu.VMEM((W,), jnp.float32),
            pltpu.VMEM_SHARED((N,), jnp.float32),
        ],
    )
    def deg_kernel(dst_hbm, zeros_hbm, ones_hbm, out_hbm, idx_v, ones_v, deg_sh):
        cid = lax.axis_index("c")
        sid = lax.axis_index("s")
        pltpu.sync_copy(ones_hbm, ones_v)

        @pl.when(sid == 0)
        def _():
            pltpu.sync_copy(zeros_hbm, deg_sh)

        plsc.subcore_barrier()
        base = (cid * NS + sid) * EPW

        @pl.loop(0, NCHUNK)
        def _(c):
            pltpu.sync_copy(dst_hbm.at[pl.ds(base + c * W, W)], idx_v)
            pltpu.sync_copy(ones_v, deg_sh.at[idx_v], add=True)

        plsc.subcore_barrier()

        @pl.when(sid == 0)
        def _():
            pltpu.sync_copy(deg_sh, out_hbm.at[cid])

    return deg_kernel(dst, zeros_n, ones_w)


def _sc_aggregate(y, src, dst):
    @functools.partial(
        pl.kernel,
        out_type=jax.ShapeDtypeStruct((NC, N, D), jnp.float32),
        mesh=_vmesh(),
        scratch_types=[
            pltpu.VMEM((W,), jnp.int32),
            pltpu.VMEM((W,), jnp.int32),
            pltpu.VMEM((W, D), jnp.float32),
            pltpu.VMEM_SHARED((N, D), jnp.float32),
        ],
    )
    def agg_kernel(y_hbm, src_hbm, dst_hbm, out_hbm, sidx_v, didx_v, rows_v, z_sh):
        cid = lax.axis_index("c")
        sid = lax.axis_index("s")
        r0 = sid * RPT
        pltpu.sync_copy(y_hbm.at[pl.ds(r0, RPT)], z_sh.at[pl.ds(r0, RPT)])
        plsc.subcore_barrier()
        base = (cid * NS + sid) * EPW

        @pl.loop(0, NCHUNK)
        def _(c):
            pltpu.sync_copy(src_hbm.at[pl.ds(base + c * W, W)], sidx_v)
            pltpu.sync_copy(dst_hbm.at[pl.ds(base + c * W, W)], didx_v)
            pltpu.sync_copy(y_hbm.at[sidx_v], rows_v)
            pltpu.sync_copy(rows_v, z_sh.at[didx_v], add=True)

        plsc.subcore_barrier()
        pltpu.sync_copy(z_sh.at[pl.ds(r0, RPT)], out_hbm.at[cid, pl.ds(r0, RPT)])

    return agg_kernel(y, src, dst)


def _dis_block(degt_blk):
    deg = degt_blk[:, 0:1] + degt_blk[:, 1:2] + 1.0
    return lax.rsqrt(deg)


def _tc_scale_matmul(x, w, degt):
    def body(x_ref, w_ref, d_ref, o_ref):
        dis = _dis_block(d_ref[...])
        o_ref[...] = dis * jnp.dot(x_ref[...], w_ref[...],
                                   preferred_element_type=jnp.float32)

    return pl.pallas_call(
        body,
        grid=(GRID,),
        in_specs=[
            pl.BlockSpec((BLK, D), lambda i: (i, 0)),
            pl.BlockSpec((D, D), lambda i: (0, 0)),
            pl.BlockSpec((BLK, 2), lambda i: (i, 0)),
        ],
        out_specs=pl.BlockSpec((BLK, D), lambda i: (i, 0)),
        out_shape=jax.ShapeDtypeStruct((N, D), jnp.float32),
    )(x, w, degt)


def _tc_mid(z, y, degt, b, w):
    def body(z_ref, y_ref, d_ref, b_ref, w_ref, o_ref):
        dis = _dis_block(d_ref[...])
        zs = z_ref[0] + z_ref[1] - y_ref[...]
        h = jnp.maximum(dis * zs + b_ref[...], 0.0)
        o_ref[...] = dis * jnp.dot(h, w_ref[...],
                                   preferred_element_type=jnp.float32)

    return pl.pallas_call(
        body,
        grid=(GRID,),
        in_specs=[
            pl.BlockSpec((NC, BLK, D), lambda i: (0, i, 0)),
            pl.BlockSpec((BLK, D), lambda i: (i, 0)),
            pl.BlockSpec((BLK, 2), lambda i: (i, 0)),
            pl.BlockSpec((D,), lambda i: (0,)),
            pl.BlockSpec((D, D), lambda i: (0, 0)),
        ],
        out_specs=pl.BlockSpec((BLK, D), lambda i: (i, 0)),
        out_shape=jax.ShapeDtypeStruct((N, D), jnp.float32),
    )(z, y, degt, b, w)


def _tc_final(z, y, degt, b, batch2, wfc, bfc):
    dout = wfc.shape[1]

    def body(z_ref, y_ref, d_ref, b_ref, bat_ref, wfc_ref, bfc_ref, o_ref,
             acc_ref, cnt_ref):
        i = pl.program_id(0)
        dis = _dis_block(d_ref[...])
        h = jnp.maximum(dis * (z_ref[0] + z_ref[1] - y_ref[...]) + b_ref[...], 0.0)
        gids = lax.broadcasted_iota(jnp.int32, (BLK, G), 1)
        oh = (bat_ref[...] == gids).astype(jnp.float32)
        pacc = lax.dot_general(oh, h, (((0,), (0,)), ((), ())),
                               preferred_element_type=jnp.float32)
        pcnt = lax.dot_general(oh, jnp.ones((BLK, D), jnp.float32),
                               (((0,), (0,)), ((), ())),
                               preferred_element_type=jnp.float32)

        @pl.when(i == 0)
        def _():
            acc_ref[...] = jnp.zeros_like(acc_ref)
            cnt_ref[...] = jnp.zeros_like(cnt_ref)

        acc_ref[...] += pacc
        cnt_ref[...] += pcnt

        @pl.when(i == GRID - 1)
        def _():
            pooled = acc_ref[...] / jnp.maximum(cnt_ref[...], 1.0)
            o_ref[...] = (jnp.dot(pooled, wfc_ref[...],
                                  preferred_element_type=jnp.float32)
                          + bfc_ref[...])

    return pl.pallas_call(
        body,
        grid=(GRID,),
        in_specs=[
            pl.BlockSpec((NC, BLK, D), lambda i: (0, i, 0)),
            pl.BlockSpec((BLK, D), lambda i: (i, 0)),
            pl.BlockSpec((BLK, 2), lambda i: (i, 0)),
            pl.BlockSpec((D,), lambda i: (0,)),
            pl.BlockSpec((BLK, 1), lambda i: (i, 0)),
            pl.BlockSpec((D, dout), lambda i: (0, 0)),
            pl.BlockSpec((dout,), lambda i: (0,)),
        ],
        out_specs=pl.BlockSpec((G, dout), lambda i: (0, 0)),
        out_shape=jax.ShapeDtypeStruct((G, dout), jnp.float32),
        scratch_shapes=[
            pltpu.VMEM((G, D), jnp.float32),
            pltpu.VMEM((G, D), jnp.float32),
        ],
    )(z, y, degt, b, batch2, wfc, bfc)


def kernel(x, edge_index, batch, W1, b1, W2, b2, Wfc, bfc):
    src = edge_index[0].astype(jnp.int32)
    dst = edge_index[1].astype(jnp.int32)
    zeros_n = jnp.zeros((N,), jnp.float32)
    ones_w = jnp.ones((W,), jnp.float32)

    degp = _sc_degree(dst, zeros_n, ones_w)
    degt = degp.T

    y1 = _tc_scale_matmul(x, W1, degt)
    z1 = _sc_aggregate(y1, src, dst)
    y2 = _tc_mid(z1, y1, degt, b1, W2)
    z2 = _sc_aggregate(y2, src, dst)
    batch2 = batch.astype(jnp.int32).reshape(N, 1)
    return _tc_final(z2, y2, degt, b2, batch2, Wfc, bfc)


# speedup vs baseline: 12.7570x; 12.7570x over previous
import functools

import jax
import jax.numpy as jnp
from jax import lax
from jax.experimental import pallas as pl
from jax.experimental.pallas import tpu as pltpu
from jax.experimental.pallas import tpu_sc as plsc

N = 10000
E = 320000
D = 128
G = 16

NC = 2
NS = 16
EPW = E // (NC * NS)
W = 80
NCHUNK = EPW // W
RPT = (N // NS) // 8 * 8
RTAIL = N - NS * RPT

BLK = 400
GRID = N // BLK


def _vmesh():
    return plsc.VectorSubcoreMesh(core_axis_name="c", subcore_axis_name="s")


def _sc_degree(dst, zeros_n, ones_w):
    @functools.partial(
        pl.kernel,
        out_type=jax.ShapeDtypeStruct((NC * N,), jnp.float32),
        mesh=_vmesh(),
        scratch_types=[
            pltpu.VMEM((W,), jnp.int32),
            pltpu.VMEM((W,), jnp.float32),
            pltpu.VMEM((N,), jnp.float32),
            pltpu.VMEM_SHARED((N,), jnp.float32),
        ],
    )
    def deg_kernel(dst_hbm, zeros_hbm, ones_hbm, out_hbm, idx_v, ones_v, tmp_v, deg_sh):
        cid = lax.axis_index("c")
        sid = lax.axis_index("s")
        pltpu.sync_copy(ones_hbm, ones_v)

        @pl.when(sid == 0)
        def _():
            pltpu.sync_copy(zeros_hbm, tmp_v)
            pltpu.sync_copy(tmp_v, deg_sh)

        plsc.subcore_barrier()
        base = (cid * NS + sid) * EPW

        @pl.loop(0, NCHUNK)
        def _(c):
            pltpu.sync_copy(dst_hbm.at[pl.ds(base + c * W, W)], idx_v)
            pltpu.sync_copy(ones_v, deg_sh.at[idx_v], add=True)

        plsc.subcore_barrier()

        @pl.when(sid == 0)
        def _():
            pltpu.sync_copy(deg_sh, tmp_v)
            pltpu.sync_copy(tmp_v, out_hbm.at[pl.ds(cid * N, N)])

    return deg_kernel(dst, zeros_n, ones_w)


def _sc_aggregate(y, src, dst):
    @functools.partial(
        pl.kernel,
        out_type=jax.ShapeDtypeStruct((NC, N, D), jnp.float32),
        mesh=_vmesh(),
        scratch_types=[
            pltpu.VMEM((W,), jnp.int32),
            pltpu.VMEM((W,), jnp.int32),
            pltpu.VMEM((W, D), jnp.float32),
            pltpu.VMEM_SHARED((N, D), jnp.float32),
        ],
    )
    def agg_kernel(y_hbm, src_hbm, dst_hbm, out_hbm, sidx_v, didx_v, rows_v, z_sh):
        cid = lax.axis_index("c")
        sid = lax.axis_index("s")
        r0 = sid * RPT
        pltpu.sync_copy(y_hbm.at[pl.ds(r0, RPT)], z_sh.at[pl.ds(r0, RPT)])

        @pl.when(sid == 0)
        def _():
            pltpu.sync_copy(y_hbm.at[pl.ds(NS * RPT, RTAIL)],
                            z_sh.at[pl.ds(NS * RPT, RTAIL)])

        plsc.subcore_barrier()
        base = (cid * NS + sid) * EPW

        @pl.loop(0, NCHUNK)
        def _(c):
            pltpu.sync_copy(src_hbm.at[pl.ds(base + c * W, W)], sidx_v)
            pltpu.sync_copy(dst_hbm.at[pl.ds(base + c * W, W)], didx_v)
            pltpu.sync_copy(y_hbm.at[sidx_v], rows_v)
            pltpu.sync_copy(rows_v, z_sh.at[didx_v], add=True)

        plsc.subcore_barrier()
        pltpu.sync_copy(z_sh.at[pl.ds(r0, RPT)], out_hbm.at[cid, pl.ds(r0, RPT)])

        @pl.when(sid == 0)
        def _():
            pltpu.sync_copy(z_sh.at[pl.ds(NS * RPT, RTAIL)],
                            out_hbm.at[cid, pl.ds(NS * RPT, RTAIL)])

    return agg_kernel(y, src, dst)


def _dis_block(degt_blk):
    deg = degt_blk[:, 0:1] + degt_blk[:, 1:2] + 1.0
    return lax.rsqrt(deg)


def _tc_scale_matmul(x, w, degt):
    def body(x_ref, w_ref, d_ref, o_ref):
        dis = _dis_block(d_ref[...])
        o_ref[...] = dis * jnp.dot(x_ref[...], w_ref[...],
                                   preferred_element_type=jnp.float32)

    return pl.pallas_call(
        body,
        grid=(GRID,),
        in_specs=[
            pl.BlockSpec((BLK, D), lambda i: (i, 0)),
            pl.BlockSpec((D, D), lambda i: (0, 0)),
            pl.BlockSpec((BLK, 2), lambda i: (i, 0)),
        ],
        out_specs=pl.BlockSpec((BLK, D), lambda i: (i, 0)),
        out_shape=jax.ShapeDtypeStruct((N, D), jnp.float32),
    )(x, w, degt)


def _tc_mid(z, y, degt, b, w):
    def body(z_ref, y_ref, d_ref, b_ref, w_ref, o_ref):
        dis = _dis_block(d_ref[...])
        zs = z_ref[0] + z_ref[1] - y_ref[...]
        h = jnp.maximum(dis * zs + b_ref[...], 0.0)
        o_ref[...] = dis * jnp.dot(h, w_ref[...],
                                   preferred_element_type=jnp.float32)

    return pl.pallas_call(
        body,
        grid=(GRID,),
        in_specs=[
            pl.BlockSpec((NC, BLK, D), lambda i: (0, i, 0)),
            pl.BlockSpec((BLK, D), lambda i: (i, 0)),
            pl.BlockSpec((BLK, 2), lambda i: (i, 0)),
            pl.BlockSpec((D,), lambda i: (0,)),
            pl.BlockSpec((D, D), lambda i: (0, 0)),
        ],
        out_specs=pl.BlockSpec((BLK, D), lambda i: (i, 0)),
        out_shape=jax.ShapeDtypeStruct((N, D), jnp.float32),
    )(z, y, degt, b, w)


def _tc_final(z, y, degt, b, batch2, wfc, bfc):
    dout = wfc.shape[1]

    def body(z_ref, y_ref, d_ref, b_ref, bat_ref, wfc_ref, bfc_ref, o_ref,
             acc_ref, cnt_ref):
        i = pl.program_id(0)
        dis = _dis_block(d_ref[...])
        h = jnp.maximum(dis * (z_ref[0] + z_ref[1] - y_ref[...]) + b_ref[...], 0.0)
        gids = lax.broadcasted_iota(jnp.int32, (BLK, G), 1)
        oh = (bat_ref[...] == gids).astype(jnp.float32)
        pacc = lax.dot_general(oh, h, (((0,), (0,)), ((), ())),
                               preferred_element_type=jnp.float32)
        pcnt = lax.dot_general(oh, jnp.ones((BLK, D), jnp.float32),
                               (((0,), (0,)), ((), ())),
                               preferred_element_type=jnp.float32)

        @pl.when(i == 0)
        def _():
            acc_ref[...] = jnp.zeros_like(acc_ref)
            cnt_ref[...] = jnp.zeros_like(cnt_ref)

        acc_ref[...] += pacc
        cnt_ref[...] += pcnt

        @pl.when(i == GRID - 1)
        def _():
            pooled = acc_ref[...] / jnp.maximum(cnt_ref[...], 1.0)
            o_ref[...] = (jnp.dot(pooled, wfc_ref[...],
                                  preferred_element_type=jnp.float32)
                          + bfc_ref[...])

    return pl.pallas_call(
        body,
        grid=(GRID,),
        in_specs=[
            pl.BlockSpec((NC, BLK, D), lambda i: (0, i, 0)),
            pl.BlockSpec((BLK, D), lambda i: (i, 0)),
            pl.BlockSpec((BLK, 2), lambda i: (i, 0)),
            pl.BlockSpec((D,), lambda i: (0,)),
            pl.BlockSpec((BLK, 1), lambda i: (i, 0)),
            pl.BlockSpec((D, dout), lambda i: (0, 0)),
            pl.BlockSpec((dout,), lambda i: (0,)),
        ],
        out_specs=pl.BlockSpec((G, dout), lambda i: (0, 0)),
        out_shape=jax.ShapeDtypeStruct((G, dout), jnp.float32),
        scratch_shapes=[
            pltpu.VMEM((G, D), jnp.float32),
            pltpu.VMEM((G, D), jnp.float32),
        ],
    )(z, y, degt, b, batch2, wfc, bfc)


def kernel(x, edge_index, batch, W1, b1, W2, b2, Wfc, bfc):
    src = edge_index[0].astype(jnp.int32)
    dst = edge_index[1].astype(jnp.int32)
    zeros_n = jnp.zeros((N,), jnp.float32)
    ones_w = jnp.ones((W,), jnp.float32)

    degp = _sc_degree(dst, zeros_n, ones_w).reshape(NC, N)
    degt = degp.T

    y1 = _tc_scale_matmul(x, W1, degt)
    z1 = _sc_aggregate(y1, src, dst)
    y2 = _tc_mid(z1, y1, degt, b1, W2)
    z2 = _sc_aggregate(y2, src, dst)
    batch2 = batch.astype(jnp.int32).reshape(N, 1)
    return _tc_final(z2, y2, degt, b2, batch2, Wfc, bfc)


# speedup vs baseline: 24.0379x; 1.8843x over previous
import functools

import jax
import jax.numpy as jnp
from jax import lax
from jax.experimental import pallas as pl
from jax.experimental.pallas import tpu as pltpu
from jax.experimental.pallas import tpu_sc as plsc

N = 10000
E = 320000
D = 128
G = 16

NC = 2
NS = 16
EPW = E // (NC * NS)
W = 80
NCHUNK = EPW // W
RPT = (N // NS) // 8 * 8
RTAIL = N - NS * RPT

BLK = 400
GRID = N // BLK


def _vmesh():
    return plsc.VectorSubcoreMesh(core_axis_name="c", subcore_axis_name="s")


NW = NC * NS


def _sc_degree(dst):
    @functools.partial(
        pl.kernel,
        out_type=jax.ShapeDtypeStruct((NW * N,), jnp.float32),
        mesh=_vmesh(),
        scratch_types=[
            pltpu.VMEM((EPW,), jnp.int32),
            pltpu.VMEM((N,), jnp.float32),
        ],
        compiler_params=pltpu.CompilerParams(needs_layout_passes=False),
    )
    def deg_kernel(dst_hbm, out_hbm, didx_all, hist):
        cid = lax.axis_index("c")
        sid = lax.axis_index("s")
        wid = cid * NS + sid
        pltpu.sync_copy(dst_hbm.at[pl.ds(wid * EPW, EPW)], didx_all)

        zeros16 = jnp.zeros((16,), jnp.float32)

        @pl.loop(0, N // 16, unroll=5)
        def _(i):
            hist[pl.ds(i * 16, 16)] = zeros16

        ones16 = jnp.full((16,), 1.0, jnp.float32)

        @pl.loop(0, EPW // 16, unroll=5)
        def _(i):
            idx = didx_all[pl.ds(i * 16, 16)]
            plsc.addupdate_scatter(hist, [idx], ones16)

        pltpu.sync_copy(hist, out_hbm.at[pl.ds(wid * N, N)])

    return deg_kernel(dst)


def _sc_aggregate(y, src, dst):
    @functools.partial(
        pl.kernel,
        out_type=jax.ShapeDtypeStruct((NC, N, D), jnp.float32),
        mesh=_vmesh(),
        scratch_types=[
            pltpu.VMEM((EPW,), jnp.int32),
            pltpu.VMEM((EPW,), jnp.int32),
            pltpu.VMEM((W,), jnp.int32),
            pltpu.VMEM((W,), jnp.int32),
            pltpu.VMEM((W, D), jnp.float32),
            pltpu.VMEM((W, D), jnp.float32),
            pltpu.VMEM_SHARED((N, D), jnp.float32),
            pltpu.SemaphoreType.DMA,
            pltpu.SemaphoreType.DMA,
            pltpu.SemaphoreType.DMA,
            pltpu.SemaphoreType.DMA,
        ],
    )
    def agg_kernel(y_hbm, src_hbm, dst_hbm, out_hbm,
                   sidx_all, didx_all, didx_a, didx_b, rows_a, rows_b, z_sh,
                   gsa, gsb, ssa, ssb):
        cid = lax.axis_index("c")
        sid = lax.axis_index("s")
        r0 = sid * RPT
        pltpu.sync_copy(y_hbm.at[pl.ds(r0, RPT)], z_sh.at[pl.ds(r0, RPT)])

        @pl.when(sid == 0)
        def _():
            pltpu.sync_copy(y_hbm.at[pl.ds(NS * RPT, RTAIL)],
                            z_sh.at[pl.ds(NS * RPT, RTAIL)])

        base = (cid * NS + sid) * EPW
        pltpu.sync_copy(src_hbm.at[pl.ds(base, EPW)], sidx_all)
        pltpu.sync_copy(dst_hbm.at[pl.ds(base, EPW)], didx_all)
        plsc.subcore_barrier()

        def load_didx(buf, j):
            @pl.loop(0, W // 16)
            def _(k):
                buf[pl.ds(k * 16, 16)] = didx_all[pl.ds(j * W + k * 16, 16)]

        def gather_start(j, rows, sem):
            pltpu.async_copy(y_hbm.at[sidx_all.at[pl.ds(j * W, W)]], rows, sem)

        def gather_wait(j, rows, sem):
            pltpu.make_async_copy(
                y_hbm.at[sidx_all.at[pl.ds(j * W, W)]], rows, sem).wait()

        def scatter_start(rows, didx, sem):
            pltpu.async_copy(rows, z_sh.at[didx], sem, add=True)

        def scatter_wait(rows, didx, sem):
            pltpu.make_async_copy(rows, z_sh.at[didx], sem).wait()

        load_didx(didx_a, 0)
        gather_start(0, rows_a, gsa)

        @pl.loop(0, NCHUNK - 1, step=2)
        def _(j):
            @pl.when(j > 0)
            def _():
                scatter_wait(rows_b, didx_b, ssb)

            load_didx(didx_b, j + 1)
            gather_start(j + 1, rows_b, gsb)
            gather_wait(j, rows_a, gsa)
            scatter_start(rows_a, didx_a, ssa)
            gather_wait(j + 1, rows_b, gsb)
            scatter_start(rows_b, didx_b, ssb)
            scatter_wait(rows_a, didx_a, ssa)
            load_didx(didx_a, j + 2)
            gather_start(j + 2, rows_a, gsa)

        scatter_wait(rows_b, didx_b, ssb)
        gather_wait(NCHUNK - 1, rows_a, gsa)
        scatter_start(rows_a, didx_a, ssa)
        scatter_wait(rows_a, didx_a, ssa)

        plsc.subcore_barrier()
        pltpu.sync_copy(z_sh.at[pl.ds(r0, RPT)], out_hbm.at[cid, pl.ds(r0, RPT)])

        @pl.when(sid == 0)
        def _():
            pltpu.sync_copy(z_sh.at[pl.ds(NS * RPT, RTAIL)],
                            out_hbm.at[cid, pl.ds(NS * RPT, RTAIL)])

    return agg_kernel(y, src, dst)


def _dis_block(degt_blk):
    deg = jnp.sum(degt_blk, axis=1, keepdims=True) + 1.0
    return lax.rsqrt(deg)


def _tc_scale_matmul(x, w, degt):
    def body(x_ref, w_ref, d_ref, o_ref):
        dis = _dis_block(d_ref[...])
        o_ref[...] = dis * jnp.dot(x_ref[...], w_ref[...],
                                   preferred_element_type=jnp.float32)

    return pl.pallas_call(
        body,
        grid=(GRID,),
        in_specs=[
            pl.BlockSpec((BLK, D), lambda i: (i, 0)),
            pl.BlockSpec((D, D), lambda i: (0, 0)),
            pl.BlockSpec((BLK, NW), lambda i: (i, 0)),
        ],
        out_specs=pl.BlockSpec((BLK, D), lambda i: (i, 0)),
        out_shape=jax.ShapeDtypeStruct((N, D), jnp.float32),
    )(x, w, degt)


def _tc_mid(z, y, degt, b, w):
    def body(z_ref, y_ref, d_ref, b_ref, w_ref, o_ref):
        dis = _dis_block(d_ref[...])
        zs = z_ref[0] + z_ref[1] - y_ref[...]
        h = jnp.maximum(dis * zs + b_ref[...], 0.0)
        o_ref[...] = dis * jnp.dot(h, w_ref[...],
                                   preferred_element_type=jnp.float32)

    return pl.pallas_call(
        body,
        grid=(GRID,),
        in_specs=[
            pl.BlockSpec((NC, BLK, D), lambda i: (0, i, 0)),
            pl.BlockSpec((BLK, D), lambda i: (i, 0)),
            pl.BlockSpec((BLK, NW), lambda i: (i, 0)),
            pl.BlockSpec((D,), lambda i: (0,)),
            pl.BlockSpec((D, D), lambda i: (0, 0)),
        ],
        out_specs=pl.BlockSpec((BLK, D), lambda i: (i, 0)),
        out_shape=jax.ShapeDtypeStruct((N, D), jnp.float32),
    )(z, y, degt, b, w)


def _tc_final(z, y, degt, b, batch2, wfc, bfc):
    dout = wfc.shape[1]

    def body(z_ref, y_ref, d_ref, b_ref, bat_ref, wfc_ref, bfc_ref, o_ref,
             acc_ref, cnt_ref):
        i = pl.program_id(0)
        dis = _dis_block(d_ref[...])
        h = jnp.maximum(dis * (z_ref[0] + z_ref[1] - y_ref[...]) + b_ref[...], 0.0)
        gids = lax.broadcasted_iota(jnp.int32, (BLK, G), 1)
        oh = (bat_ref[...] == gids).astype(jnp.float32)
        pacc = lax.dot_general(oh, h, (((0,), (0,)), ((), ())),
                               preferred_element_type=jnp.float32)
        pcnt = lax.dot_general(oh, jnp.ones((BLK, D), jnp.float32),
                               (((0,), (0,)), ((), ())),
                               preferred_element_type=jnp.float32)

        @pl.when(i == 0)
        def _():
            acc_ref[...] = jnp.zeros_like(acc_ref)
            cnt_ref[...] = jnp.zeros_like(cnt_ref)

        acc_ref[...] += pacc
        cnt_ref[...] += pcnt

        @pl.when(i == GRID - 1)
        def _():
            pooled = acc_ref[...] / jnp.maximum(cnt_ref[...], 1.0)
            o_ref[...] = (jnp.dot(pooled, wfc_ref[...],
                                  preferred_element_type=jnp.float32)
                          + bfc_ref[...])

    return pl.pallas_call(
        body,
        grid=(GRID,),
        in_specs=[
            pl.BlockSpec((NC, BLK, D), lambda i: (0, i, 0)),
            pl.BlockSpec((BLK, D), lambda i: (i, 0)),
            pl.BlockSpec((BLK, NW), lambda i: (i, 0)),
            pl.BlockSpec((D,), lambda i: (0,)),
            pl.BlockSpec((BLK, 1), lambda i: (i, 0)),
            pl.BlockSpec((D, dout), lambda i: (0, 0)),
            pl.BlockSpec((dout,), lambda i: (0,)),
        ],
        out_specs=pl.BlockSpec((G, dout), lambda i: (0, 0)),
        out_shape=jax.ShapeDtypeStruct((G, dout), jnp.float32),
        scratch_shapes=[
            pltpu.VMEM((G, D), jnp.float32),
            pltpu.VMEM((G, D), jnp.float32),
        ],
    )(z, y, degt, b, batch2, wfc, bfc)


def kernel(x, edge_index, batch, W1, b1, W2, b2, Wfc, bfc):
    src = edge_index[0].astype(jnp.int32)
    dst = edge_index[1].astype(jnp.int32)

    degp = _sc_degree(dst).reshape(NW, N)
    degt = degp.T

    y1 = _tc_scale_matmul(x, W1, degt)
    z1 = _sc_aggregate(y1, src, dst)
    y2 = _tc_mid(z1, y1, degt, b1, W2)
    z2 = _sc_aggregate(y2, src, dst)
    batch2 = batch.astype(jnp.int32).reshape(N, 1)
    return _tc_final(z2, y2, degt, b2, batch2, Wfc, bfc)


# speedup vs baseline: 26.4536x; 1.1005x over previous
import functools

import jax
import jax.numpy as jnp
from jax import lax
from jax.experimental import pallas as pl
from jax.experimental.pallas import tpu as pltpu
from jax.experimental.pallas import tpu_sc as plsc

N = 10000
E = 320000
D = 128
G = 16

NC = 2
NS = 16
EPW = E // (NC * NS)
W = 80
NCHUNK = EPW // W
RPT = (N // NS) // 8 * 8
RTAIL = N - NS * RPT

BLK = 400
GRID = N // BLK


def _vmesh():
    return plsc.VectorSubcoreMesh(core_axis_name="c", subcore_axis_name="s")


NW = NC * NS


def _sc_degree(dst):
    @functools.partial(
        pl.kernel,
        out_type=jax.ShapeDtypeStruct((NW * N,), jnp.float32),
        mesh=_vmesh(),
        scratch_types=[
            pltpu.VMEM((EPW,), jnp.int32),
            pltpu.VMEM((N,), jnp.float32),
        ],
        compiler_params=pltpu.CompilerParams(needs_layout_passes=False),
    )
    def deg_kernel(dst_hbm, out_hbm, didx_all, hist):
        cid = lax.axis_index("c")
        sid = lax.axis_index("s")
        wid = cid * NS + sid
        pltpu.sync_copy(dst_hbm.at[pl.ds(wid * EPW, EPW)], didx_all)

        zeros16 = jnp.zeros((16,), jnp.float32)

        @pl.loop(0, N // 16, unroll=5)
        def _(i):
            hist[pl.ds(i * 16, 16)] = zeros16

        ones16 = jnp.full((16,), 1.0, jnp.float32)

        @pl.loop(0, EPW // 16, unroll=5)
        def _(i):
            idx = didx_all[pl.ds(i * 16, 16)]
            plsc.addupdate_scatter(hist, [idx], ones16)

        pltpu.sync_copy(hist, out_hbm.at[pl.ds(wid * N, N)])

    return deg_kernel(dst)


def _sc_aggregate(y, src, dst):
    @functools.partial(
        pl.kernel,
        out_type=jax.ShapeDtypeStruct((NC, N, D), jnp.float32),
        mesh=_vmesh(),
        scratch_types=[
            [pltpu.VMEM((W,), jnp.int32)] * 4,
            [pltpu.VMEM((W,), jnp.int32)] * 4,
            [pltpu.VMEM((W, D), jnp.float32)] * 4,
            pltpu.VMEM_SHARED((N, D), jnp.float32),
            [pltpu.SemaphoreType.DMA] * 4,
            [pltpu.SemaphoreType.DMA] * 4,
            [pltpu.SemaphoreType.DMA] * 4,
            [pltpu.SemaphoreType.DMA] * 4,
        ],
    )
    def agg_kernel(y_hbm, src_hbm, dst_hbm, out_hbm,
                   sidx_r, didx_r, rows_r, z_sh, sisem, disem, gsem, ssem):
        cid = lax.axis_index("c")
        sid = lax.axis_index("s")
        r0 = sid * RPT
        pltpu.sync_copy(y_hbm.at[pl.ds(r0, RPT)], z_sh.at[pl.ds(r0, RPT)])

        @pl.when(sid == 0)
        def _():
            pltpu.sync_copy(y_hbm.at[pl.ds(NS * RPT, RTAIL)],
                            z_sh.at[pl.ds(NS * RPT, RTAIL)])

        base = (cid * NS + sid) * EPW
        plsc.subcore_barrier()

        def idx_start(q, b):
            pltpu.async_copy(src_hbm.at[pl.ds(base + q * W, W)],
                             sidx_r[b], sisem[b])
            pltpu.async_copy(dst_hbm.at[pl.ds(base + q * W, W)],
                             didx_r[b], disem[b])

        def idx_wait(q, b):
            pltpu.make_async_copy(src_hbm.at[pl.ds(base + q * W, W)],
                                  sidx_r[b], sisem[b]).wait()
            pltpu.make_async_copy(dst_hbm.at[pl.ds(base + q * W, W)],
                                  didx_r[b], disem[b]).wait()

        def gather_start(b):
            pltpu.async_copy(y_hbm.at[sidx_r[b]], rows_r[b], gsem[b])

        def gather_wait(b):
            pltpu.make_async_copy(y_hbm.at[sidx_r[b]], rows_r[b],
                                  gsem[b]).wait()

        def scatter_start(b):
            pltpu.async_copy(rows_r[b], z_sh.at[didx_r[b]], ssem[b], add=True)

        def scatter_wait(b):
            pltpu.make_async_copy(rows_r[b], z_sh.at[didx_r[b]],
                                  ssem[b]).wait()

        NB = 4
        for b in range(NB):
            idx_start(b, b)

        @pl.loop(0, NCHUNK - 1, step=NB)
        def _(j):
            for b in range(NB):
                idx_wait(j + b, b)
                gather_start(b)
            for b in range(NB):
                gather_wait(b)
                scatter_start(b)
            for b in range(NB):
                scatter_wait(b)
                pl.when(j + b + NB < NCHUNK)(
                    functools.partial(idx_start, j + b + NB, b))

        idx_wait(NCHUNK - 1, 0)
        gather_start(0)
        gather_wait(0)
        scatter_start(0)
        scatter_wait(0)

        plsc.subcore_barrier()
        pltpu.sync_copy(z_sh.at[pl.ds(r0, RPT)], out_hbm.at[cid, pl.ds(r0, RPT)])

        @pl.when(sid == 0)
        def _():
            pltpu.sync_copy(z_sh.at[pl.ds(NS * RPT, RTAIL)],
                            out_hbm.at[cid, pl.ds(NS * RPT, RTAIL)])

    return agg_kernel(y, src, dst)


def _dis_block(degt_blk):
    deg = jnp.sum(degt_blk, axis=1, keepdims=True) + 1.0
    return lax.rsqrt(deg)


def _tc_scale_matmul(x, w, degt):
    def body(x_ref, w_ref, d_ref, o_ref):
        dis = _dis_block(d_ref[...])
        o_ref[...] = dis * jnp.dot(x_ref[...], w_ref[...],
                                   preferred_element_type=jnp.float32)

    return pl.pallas_call(
        body,
        grid=(GRID,),
        in_specs=[
            pl.BlockSpec((BLK, D), lambda i: (i, 0)),
            pl.BlockSpec((D, D), lambda i: (0, 0)),
            pl.BlockSpec((BLK, NW), lambda i: (i, 0)),
        ],
        out_specs=pl.BlockSpec((BLK, D), lambda i: (i, 0)),
        out_shape=jax.ShapeDtypeStruct((N, D), jnp.float32),
    )(x, w, degt)


def _tc_mid(z, y, degt, b, w):
    def body(z_ref, y_ref, d_ref, b_ref, w_ref, o_ref):
        dis = _dis_block(d_ref[...])
        zs = z_ref[0] + z_ref[1] - y_ref[...]
        h = jnp.maximum(dis * zs + b_ref[...], 0.0)
        o_ref[...] = dis * jnp.dot(h, w_ref[...],
                                   preferred_element_type=jnp.float32)

    return pl.pallas_call(
        body,
        grid=(GRID,),
        in_specs=[
            pl.BlockSpec((NC, BLK, D), lambda i: (0, i, 0)),
            pl.BlockSpec((BLK, D), lambda i: (i, 0)),
            pl.BlockSpec((BLK, NW), lambda i: (i, 0)),
            pl.BlockSpec((D,), lambda i: (0,)),
            pl.BlockSpec((D, D), lambda i: (0, 0)),
        ],
        out_specs=pl.BlockSpec((BLK, D), lambda i: (i, 0)),
        out_shape=jax.ShapeDtypeStruct((N, D), jnp.float32),
    )(z, y, degt, b, w)


def _tc_final(z, y, degt, b, batch2, wfc, bfc):
    dout = wfc.shape[1]

    def body(z_ref, y_ref, d_ref, b_ref, bat_ref, wfc_ref, bfc_ref, o_ref,
             acc_ref, cnt_ref):
        i = pl.program_id(0)
        dis = _dis_block(d_ref[...])
        h = jnp.maximum(dis * (z_ref[0] + z_ref[1] - y_ref[...]) + b_ref[...], 0.0)
        gids = lax.broadcasted_iota(jnp.int32, (BLK, G), 1)
        oh = (bat_ref[...] == gids).astype(jnp.float32)
        pacc = lax.dot_general(oh, h, (((0,), (0,)), ((), ())),
                               preferred_element_type=jnp.float32)
        pcnt = lax.dot_general(oh, jnp.ones((BLK, D), jnp.float32),
                               (((0,), (0,)), ((), ())),
                               preferred_element_type=jnp.float32)

        @pl.when(i == 0)
        def _():
            acc_ref[...] = jnp.zeros_like(acc_ref)
            cnt_ref[...] = jnp.zeros_like(cnt_ref)

        acc_ref[...] += pacc
        cnt_ref[...] += pcnt

        @pl.when(i == GRID - 1)
        def _():
            pooled = acc_ref[...] / jnp.maximum(cnt_ref[...], 1.0)
            o_ref[...] = (jnp.dot(pooled, wfc_ref[...],
                                  preferred_element_type=jnp.float32)
                          + bfc_ref[...])

    return pl.pallas_call(
        body,
        grid=(GRID,),
        in_specs=[
            pl.BlockSpec((NC, BLK, D), lambda i: (0, i, 0)),
            pl.BlockSpec((BLK, D), lambda i: (i, 0)),
            pl.BlockSpec((BLK, NW), lambda i: (i, 0)),
            pl.BlockSpec((D,), lambda i: (0,)),
            pl.BlockSpec((BLK, 1), lambda i: (i, 0)),
            pl.BlockSpec((D, dout), lambda i: (0, 0)),
            pl.BlockSpec((dout,), lambda i: (0,)),
        ],
        out_specs=pl.BlockSpec((G, dout), lambda i: (0, 0)),
        out_shape=jax.ShapeDtypeStruct((G, dout), jnp.float32),
        scratch_shapes=[
            pltpu.VMEM((G, D), jnp.float32),
            pltpu.VMEM((G, D), jnp.float32),
        ],
    )(z, y, degt, b, batch2, wfc, bfc)


def kernel(x, edge_index, batch, W1, b1, W2, b2, Wfc, bfc):
    src = edge_index[0].astype(jnp.int32)
    dst = edge_index[1].astype(jnp.int32)

    degp = _sc_degree(dst).reshape(NW, N)
    degt = degp.T

    y1 = _tc_scale_matmul(x, W1, degt)
    z1 = _sc_aggregate(y1, src, dst)
    y2 = _tc_mid(z1, y1, degt, b1, W2)
    z2 = _sc_aggregate(y2, src, dst)
    batch2 = batch.astype(jnp.int32).reshape(N, 1)
    return _tc_final(z2, y2, degt, b2, batch2, Wfc, bfc)


# speedup vs baseline: 27.7429x; 1.0487x over previous
import functools

import jax
import jax.numpy as jnp
from jax import lax
from jax.experimental import pallas as pl
from jax.experimental.pallas import tpu as pltpu
from jax.experimental.pallas import tpu_sc as plsc

N = 10000
E = 320000
D = 128
G = 16

NC = 2
NS = 16
EPW = E // (NC * NS)
W = 128
NCHUNK = EPW // W
ETAIL = EPW - NCHUNK * W
NBUF = 3
RPT = (N // NS) // 8 * 8
RTAIL = N - NS * RPT

BLK = 2000
GRID = N // BLK


def _vmesh():
    return plsc.VectorSubcoreMesh(core_axis_name="c", subcore_axis_name="s")


NW = NC * NS


def _sc_degree(dst):
    @functools.partial(
        pl.kernel,
        out_type=jax.ShapeDtypeStruct((NW * N,), jnp.float32),
        mesh=_vmesh(),
        scratch_types=[
            pltpu.VMEM((EPW,), jnp.int32),
            pltpu.VMEM((N,), jnp.float32),
        ],
        compiler_params=pltpu.CompilerParams(needs_layout_passes=False),
    )
    def deg_kernel(dst_hbm, out_hbm, didx_all, hist):
        cid = lax.axis_index("c")
        sid = lax.axis_index("s")
        wid = cid * NS + sid
        pltpu.sync_copy(dst_hbm.at[pl.ds(wid * EPW, EPW)], didx_all)

        zeros16 = jnp.zeros((16,), jnp.float32)

        @pl.loop(0, N // 16, unroll=5)
        def _(i):
            hist[pl.ds(i * 16, 16)] = zeros16

        ones16 = jnp.full((16,), 1.0, jnp.float32)

        @pl.loop(0, EPW // 16, unroll=5)
        def _(i):
            idx = didx_all[pl.ds(i * 16, 16)]
            plsc.addupdate_scatter(hist, [idx], ones16)

        pltpu.sync_copy(hist, out_hbm.at[pl.ds(wid * N, N)])

    return deg_kernel(dst)


def _sc_aggregate(y, src, dst):
    @functools.partial(
        pl.kernel,
        out_type=jax.ShapeDtypeStruct((NC, N, D), jnp.float32),
        mesh=_vmesh(),
        scratch_types=[
            [pltpu.VMEM((W,), jnp.int32)] * NBUF,
            [pltpu.VMEM((W,), jnp.int32)] * NBUF,
            [pltpu.VMEM((W, D), jnp.float32)] * NBUF,
            pltpu.VMEM((ETAIL,), jnp.int32),
            pltpu.VMEM((ETAIL,), jnp.int32),
            pltpu.VMEM_SHARED((N, D), jnp.float32),
            [pltpu.SemaphoreType.DMA] * NBUF,
            [pltpu.SemaphoreType.DMA] * NBUF,
            [pltpu.SemaphoreType.DMA] * NBUF,
            [pltpu.SemaphoreType.DMA] * NBUF,
        ],
    )
    def agg_kernel(y_hbm, src_hbm, dst_hbm, out_hbm,
                   sidx_r, didx_r, rows_r, tsidx, tdidx, z_sh,
                   sisem, disem, gsem, ssem):
        cid = lax.axis_index("c")
        sid = lax.axis_index("s")
        r0 = sid * RPT
        pltpu.sync_copy(y_hbm.at[pl.ds(r0, RPT)], z_sh.at[pl.ds(r0, RPT)])

        @pl.when(sid == 0)
        def _():
            pltpu.sync_copy(y_hbm.at[pl.ds(NS * RPT, RTAIL)],
                            z_sh.at[pl.ds(NS * RPT, RTAIL)])

        base = (cid * NS + sid) * EPW
        plsc.subcore_barrier()

        def idx_start(q, b):
            pltpu.async_copy(src_hbm.at[pl.ds(base + q * W, W)],
                             sidx_r[b], sisem[b])
            pltpu.async_copy(dst_hbm.at[pl.ds(base + q * W, W)],
                             didx_r[b], disem[b])

        def idx_wait(q, b):
            pltpu.make_async_copy(src_hbm.at[pl.ds(base + q * W, W)],
                                  sidx_r[b], sisem[b]).wait()
            pltpu.make_async_copy(dst_hbm.at[pl.ds(base + q * W, W)],
                                  didx_r[b], disem[b]).wait()

        def gather_start(b):
            pltpu.async_copy(y_hbm.at[sidx_r[b]], rows_r[b], gsem[b])

        def gather_wait(b):
            pltpu.make_async_copy(y_hbm.at[sidx_r[b]], rows_r[b],
                                  gsem[b]).wait()

        def scatter_start(b):
            pltpu.async_copy(rows_r[b], z_sh.at[didx_r[b]], ssem[b], add=True)

        def scatter_wait(b):
            pltpu.make_async_copy(rows_r[b], z_sh.at[didx_r[b]],
                                  ssem[b]).wait()

        for b in range(NBUF):
            idx_start(b, b)

        @pl.loop(0, NCHUNK, step=NBUF)
        def _(j):
            for b in range(NBUF):
                idx_wait(j + b, b)
                gather_start(b)
            for b in range(NBUF):
                gather_wait(b)
                scatter_start(b)
            for b in range(NBUF):
                scatter_wait(b)
                pl.when(j + b + NBUF < NCHUNK)(
                    functools.partial(idx_start, j + b + NBUF, b))

        tb = base + NCHUNK * W
        pltpu.sync_copy(src_hbm.at[pl.ds(tb, ETAIL)], tsidx)
        pltpu.sync_copy(dst_hbm.at[pl.ds(tb, ETAIL)], tdidx)
        trows = rows_r[0].at[pl.ds(0, ETAIL)]
        pltpu.sync_copy(y_hbm.at[tsidx], trows)
        pltpu.sync_copy(trows, z_sh.at[tdidx], add=True)

        plsc.subcore_barrier()
        pltpu.sync_copy(z_sh.at[pl.ds(r0, RPT)], out_hbm.at[cid, pl.ds(r0, RPT)])

        @pl.when(sid == 0)
        def _():
            pltpu.sync_copy(z_sh.at[pl.ds(NS * RPT, RTAIL)],
                            out_hbm.at[cid, pl.ds(NS * RPT, RTAIL)])

    return agg_kernel(y, src, dst)


def _dis_block(degt_blk):
    deg = jnp.sum(degt_blk, axis=1, keepdims=True) + 1.0
    return lax.rsqrt(deg)


def _tc_scale_matmul(x, w, degt):
    def body(x_ref, w_ref, d_ref, o_ref):
        dis = _dis_block(d_ref[...])
        o_ref[...] = dis * jnp.dot(x_ref[...], w_ref[...],
                                   preferred_element_type=jnp.float32)

    return pl.pallas_call(
        body,
        grid=(GRID,),
        in_specs=[
            pl.BlockSpec((BLK, D), lambda i: (i, 0)),
            pl.BlockSpec((D, D), lambda i: (0, 0)),
            pl.BlockSpec((BLK, NW), lambda i: (i, 0)),
        ],
        out_specs=pl.BlockSpec((BLK, D), lambda i: (i, 0)),
        out_shape=jax.ShapeDtypeStruct((N, D), jnp.float32),
    )(x, w, degt)


def _tc_mid(z, y, degt, b, w):
    def body(z_ref, y_ref, d_ref, b_ref, w_ref, o_ref):
        dis = _dis_block(d_ref[...])
        zs = z_ref[0] + z_ref[1] - y_ref[...]
        h = jnp.maximum(dis * zs + b_ref[...], 0.0)
        o_ref[...] = dis * jnp.dot(h, w_ref[...],
                                   preferred_element_type=jnp.float32)

    return pl.pallas_call(
        body,
        grid=(GRID,),
        in_specs=[
            pl.BlockSpec((NC, BLK, D), lambda i: (0, i, 0)),
            pl.BlockSpec((BLK, D), lambda i: (i, 0)),
            pl.BlockSpec((BLK, NW), lambda i: (i, 0)),
            pl.BlockSpec((D,), lambda i: (0,)),
            pl.BlockSpec((D, D), lambda i: (0, 0)),
        ],
        out_specs=pl.BlockSpec((BLK, D), lambda i: (i, 0)),
        out_shape=jax.ShapeDtypeStruct((N, D), jnp.float32),
    )(z, y, degt, b, w)


def _tc_final(z, y, degt, b, batch2, wfc, bfc):
    dout = wfc.shape[1]

    def body(z_ref, y_ref, d_ref, b_ref, bat_ref, wfc_ref, bfc_ref, o_ref,
             acc_ref, cnt_ref):
        i = pl.program_id(0)
        dis = _dis_block(d_ref[...])
        h = jnp.maximum(dis * (z_ref[0] + z_ref[1] - y_ref[...]) + b_ref[...], 0.0)
        gids = lax.broadcasted_iota(jnp.int32, (BLK, G), 1)
        oh = (bat_ref[...] == gids).astype(jnp.float32)
        pacc = lax.dot_general(oh, h, (((0,), (0,)), ((), ())),
                               preferred_element_type=jnp.float32)
        pcnt = lax.dot_general(oh, jnp.ones((BLK, D), jnp.float32),
                               (((0,), (0,)), ((), ())),
                               preferred_element_type=jnp.float32)

        @pl.when(i == 0)
        def _():
            acc_ref[...] = jnp.zeros_like(acc_ref)
            cnt_ref[...] = jnp.zeros_like(cnt_ref)

        acc_ref[...] += pacc
        cnt_ref[...] += pcnt

        @pl.when(i == GRID - 1)
        def _():
            pooled = acc_ref[...] / jnp.maximum(cnt_ref[...], 1.0)
            o_ref[...] = (jnp.dot(pooled, wfc_ref[...],
                                  preferred_element_type=jnp.float32)
                          + bfc_ref[...])

    return pl.pallas_call(
        body,
        grid=(GRID,),
        in_specs=[
            pl.BlockSpec((NC, BLK, D), lambda i: (0, i, 0)),
            pl.BlockSpec((BLK, D), lambda i: (i, 0)),
            pl.BlockSpec((BLK, NW), lambda i: (i, 0)),
            pl.BlockSpec((D,), lambda i: (0,)),
            pl.BlockSpec((BLK, 1), lambda i: (i, 0)),
            pl.BlockSpec((D, dout), lambda i: (0, 0)),
            pl.BlockSpec((dout,), lambda i: (0,)),
        ],
        out_specs=pl.BlockSpec((G, dout), lambda i: (0, 0)),
        out_shape=jax.ShapeDtypeStruct((G, dout), jnp.float32),
        scratch_shapes=[
            pltpu.VMEM((G, D), jnp.float32),
            pltpu.VMEM((G, D), jnp.float32),
        ],
    )(z, y, degt, b, batch2, wfc, bfc)


def kernel(x, edge_index, batch, W1, b1, W2, b2, Wfc, bfc):
    src = edge_index[0].astype(jnp.int32)
    dst = edge_index[1].astype(jnp.int32)

    degp = _sc_degree(dst).reshape(NW, N)
    degt = degp.T

    y1 = _tc_scale_matmul(x, W1, degt)
    z1 = _sc_aggregate(y1, src, dst)
    y2 = _tc_mid(z1, y1, degt, b1, W2)
    z2 = _sc_aggregate(y2, src, dst)
    batch2 = batch.astype(jnp.int32).reshape(N, 1)
    return _tc_final(z2, y2, degt, b2, batch2, Wfc, bfc)


# speedup vs baseline: 28.8418x; 1.0396x over previous
import functools

import jax
import jax.numpy as jnp
from jax import lax
from jax.experimental import pallas as pl
from jax.experimental.pallas import tpu as pltpu
from jax.experimental.pallas import tpu_sc as plsc

N = 10000
E = 320000
D = 128
G = 16

NC = 2
NS = 16
EPW = E // (NC * NS)
W = 96
NCHUNK = EPW // W
ETAIL = EPW - NCHUNK * W
NBUF = 4
RPT = (N // NS) // 8 * 8
RTAIL = N - NS * RPT

BLK = 2000
GRID = N // BLK


def _vmesh():
    return plsc.VectorSubcoreMesh(core_axis_name="c", subcore_axis_name="s")


NW = NC * NS


def _sc_degree(dst):
    @functools.partial(
        pl.kernel,
        out_type=jax.ShapeDtypeStruct((NW * N,), jnp.float32),
        mesh=_vmesh(),
        scratch_types=[
            pltpu.VMEM((EPW,), jnp.int32),
            pltpu.VMEM((N,), jnp.float32),
        ],
        compiler_params=pltpu.CompilerParams(needs_layout_passes=False),
    )
    def deg_kernel(dst_hbm, out_hbm, didx_all, hist):
        cid = lax.axis_index("c")
        sid = lax.axis_index("s")
        wid = cid * NS + sid
        pltpu.sync_copy(dst_hbm.at[pl.ds(wid * EPW, EPW)], didx_all)

        zeros16 = jnp.zeros((16,), jnp.float32)

        @pl.loop(0, N // 16, unroll=5)
        def _(i):
            hist[pl.ds(i * 16, 16)] = zeros16

        ones16 = jnp.full((16,), 1.0, jnp.float32)

        @pl.loop(0, EPW // 16, unroll=5)
        def _(i):
            idx = didx_all[pl.ds(i * 16, 16)]
            plsc.addupdate_scatter(hist, [idx], ones16)

        pltpu.sync_copy(hist, out_hbm.at[pl.ds(wid * N, N)])

    return deg_kernel(dst)


def _sc_aggregate(y, src, dst):
    @functools.partial(
        pl.kernel,
        out_type=jax.ShapeDtypeStruct((NC, N, D), jnp.float32),
        mesh=_vmesh(),
        scratch_types=[
            [pltpu.VMEM((W,), jnp.int32)] * NBUF,
            [pltpu.VMEM((W,), jnp.int32)] * NBUF,
            [pltpu.VMEM((W, D), jnp.float32)] * NBUF,
            pltpu.VMEM((ETAIL,), jnp.int32),
            pltpu.VMEM((ETAIL,), jnp.int32),
            pltpu.VMEM_SHARED((N, D), jnp.float32),
            [pltpu.SemaphoreType.DMA] * NBUF,
            [pltpu.SemaphoreType.DMA] * NBUF,
            [pltpu.SemaphoreType.DMA] * NBUF,
            [pltpu.SemaphoreType.DMA] * NBUF,
        ],
    )
    def agg_kernel(y_hbm, src_hbm, dst_hbm, out_hbm,
                   sidx_r, didx_r, rows_r, tsidx, tdidx, z_sh,
                   sisem, disem, gsem, ssem):
        cid = lax.axis_index("c")
        sid = lax.axis_index("s")
        r0 = sid * RPT
        pltpu.sync_copy(y_hbm.at[pl.ds(r0, RPT)], z_sh.at[pl.ds(r0, RPT)])

        @pl.when(sid == 0)
        def _():
            pltpu.sync_copy(y_hbm.at[pl.ds(NS * RPT, RTAIL)],
                            z_sh.at[pl.ds(NS * RPT, RTAIL)])

        base = (cid * NS + sid) * EPW
        plsc.subcore_barrier()

        def idx_start(q, b):
            pltpu.async_copy(src_hbm.at[pl.ds(base + q * W, W)],
                             sidx_r[b], sisem[b])
            pltpu.async_copy(dst_hbm.at[pl.ds(base + q * W, W)],
                             didx_r[b], disem[b])

        def idx_wait(q, b):
            pltpu.make_async_copy(src_hbm.at[pl.ds(base + q * W, W)],
                                  sidx_r[b], sisem[b]).wait()
            pltpu.make_async_copy(dst_hbm.at[pl.ds(base + q * W, W)],
                                  didx_r[b], disem[b]).wait()

        def gather_start(b):
            pltpu.async_copy(y_hbm.at[sidx_r[b]], rows_r[b], gsem[b])

        def gather_wait(b):
            pltpu.make_async_copy(y_hbm.at[sidx_r[b]], rows_r[b],
                                  gsem[b]).wait()

        def scatter_start(b):
            pltpu.async_copy(rows_r[b], z_sh.at[didx_r[b]], ssem[b], add=True)

        def scatter_wait(b):
            pltpu.make_async_copy(rows_r[b], z_sh.at[didx_r[b]],
                                  ssem[b]).wait()

        for b in range(NBUF):
            idx_start(b, b)

        @pl.loop(0, NCHUNK, step=NBUF)
        def _(j):
            for b in range(NBUF):
                idx_wait(j + b, b)
                gather_start(b)
            for b in range(NBUF):
                gather_wait(b)
                scatter_start(b)
            for b in range(NBUF):
                scatter_wait(b)
                pl.when(j + b + NBUF < NCHUNK)(
                    functools.partial(idx_start, j + b + NBUF, b))

        tb = base + NCHUNK * W
        pltpu.sync_copy(src_hbm.at[pl.ds(tb, ETAIL)], tsidx)
        pltpu.sync_copy(dst_hbm.at[pl.ds(tb, ETAIL)], tdidx)
        trows = rows_r[0].at[pl.ds(0, ETAIL)]
        pltpu.sync_copy(y_hbm.at[tsidx], trows)
        pltpu.sync_copy(trows, z_sh.at[tdidx], add=True)

        plsc.subcore_barrier()
        pltpu.sync_copy(z_sh.at[pl.ds(r0, RPT)], out_hbm.at[cid, pl.ds(r0, RPT)])

        @pl.when(sid == 0)
        def _():
            pltpu.sync_copy(z_sh.at[pl.ds(NS * RPT, RTAIL)],
                            out_hbm.at[cid, pl.ds(NS * RPT, RTAIL)])

    return agg_kernel(y, src, dst)


def _dis_block(degt_blk):
    deg = jnp.sum(degt_blk, axis=1, keepdims=True) + 1.0
    return lax.rsqrt(deg)


def _tc_scale_matmul(x, w, degt):
    def body(x_ref, w_ref, d_ref, o_ref):
        dis = _dis_block(d_ref[...])
        o_ref[...] = dis * jnp.dot(x_ref[...], w_ref[...],
                                   preferred_element_type=jnp.float32)

    return pl.pallas_call(
        body,
        grid=(GRID,),
        in_specs=[
            pl.BlockSpec((BLK, D), lambda i: (i, 0)),
            pl.BlockSpec((D, D), lambda i: (0, 0)),
            pl.BlockSpec((BLK, NW), lambda i: (i, 0)),
        ],
        out_specs=pl.BlockSpec((BLK, D), lambda i: (i, 0)),
        out_shape=jax.ShapeDtypeStruct((N, D), jnp.float32),
    )(x, w, degt)


def _tc_mid(z, y, degt, b, w):
    def body(z_ref, y_ref, d_ref, b_ref, w_ref, o_ref):
        dis = _dis_block(d_ref[...])
        zs = z_ref[0] + z_ref[1] - y_ref[...]
        h = jnp.maximum(dis * zs + b_ref[...], 0.0)
        o_ref[...] = dis * jnp.dot(h, w_ref[...],
                                   preferred_element_type=jnp.float32)

    return pl.pallas_call(
        body,
        grid=(GRID,),
        in_specs=[
            pl.BlockSpec((NC, BLK, D), lambda i: (0, i, 0)),
            pl.BlockSpec((BLK, D), lambda i: (i, 0)),
            pl.BlockSpec((BLK, NW), lambda i: (i, 0)),
            pl.BlockSpec((D,), lambda i: (0,)),
            pl.BlockSpec((D, D), lambda i: (0, 0)),
        ],
        out_specs=pl.BlockSpec((BLK, D), lambda i: (i, 0)),
        out_shape=jax.ShapeDtypeStruct((N, D), jnp.float32),
    )(z, y, degt, b, w)


def _tc_final(z, y, degt, b, batch2, wfc, bfc):
    dout = wfc.shape[1]

    def body(z_ref, y_ref, d_ref, b_ref, bat_ref, wfc_ref, bfc_ref, o_ref,
             acc_ref, cnt_ref):
        i = pl.program_id(0)
        dis = _dis_block(d_ref[...])
        h = jnp.maximum(dis * (z_ref[0] + z_ref[1] - y_ref[...]) + b_ref[...], 0.0)
        gids = lax.broadcasted_iota(jnp.int32, (BLK, G), 1)
        oh = (bat_ref[...] == gids).astype(jnp.float32)
        pacc = lax.dot_general(oh, h, (((0,), (0,)), ((), ())),
                               preferred_element_type=jnp.float32)
        pcnt = lax.dot_general(oh, jnp.ones((BLK, D), jnp.float32),
                               (((0,), (0,)), ((), ())),
                               preferred_element_type=jnp.float32)

        @pl.when(i == 0)
        def _():
            acc_ref[...] = jnp.zeros_like(acc_ref)
            cnt_ref[...] = jnp.zeros_like(cnt_ref)

        acc_ref[...] += pacc
        cnt_ref[...] += pcnt

        @pl.when(i == GRID - 1)
        def _():
            pooled = acc_ref[...] / jnp.maximum(cnt_ref[...], 1.0)
            o_ref[...] = (jnp.dot(pooled, wfc_ref[...],
                                  preferred_element_type=jnp.float32)
                          + bfc_ref[...])

    return pl.pallas_call(
        body,
        grid=(GRID,),
        in_specs=[
            pl.BlockSpec((NC, BLK, D), lambda i: (0, i, 0)),
            pl.BlockSpec((BLK, D), lambda i: (i, 0)),
            pl.BlockSpec((BLK, NW), lambda i: (i, 0)),
            pl.BlockSpec((D,), lambda i: (0,)),
            pl.BlockSpec((BLK, 1), lambda i: (i, 0)),
            pl.BlockSpec((D, dout), lambda i: (0, 0)),
            pl.BlockSpec((dout,), lambda i: (0,)),
        ],
        out_specs=pl.BlockSpec((G, dout), lambda i: (0, 0)),
        out_shape=jax.ShapeDtypeStruct((G, dout), jnp.float32),
        scratch_shapes=[
            pltpu.VMEM((G, D), jnp.float32),
            pltpu.VMEM((G, D), jnp.float32),
        ],
    )(z, y, degt, b, batch2, wfc, bfc)


def kernel(x, edge_index, batch, W1, b1, W2, b2, Wfc, bfc):
    src = edge_index[0].astype(jnp.int32)
    dst = edge_index[1].astype(jnp.int32)

    degp = _sc_degree(dst).reshape(NW, N)
    degt = degp.T

    y1 = _tc_scale_matmul(x, W1, degt)
    z1 = _sc_aggregate(y1, src, dst)
    y2 = _tc_mid(z1, y1, degt, b1, W2)
    z2 = _sc_aggregate(y2, src, dst)
    batch2 = batch.astype(jnp.int32).reshape(N, 1)
    return _tc_final(z2, y2, degt, b2, batch2, Wfc, bfc)


# speedup vs baseline: 29.0999x; 1.0089x over previous
import functools

import jax
import jax.numpy as jnp
from jax import lax
from jax.experimental import pallas as pl
from jax.experimental.pallas import tpu as pltpu
from jax.experimental.pallas import tpu_sc as plsc

N = 10000
E = 320000
D = 128
G = 16

NC = 2
NS = 16
EPW = E // (NC * NS)
W = 64
NCHUNK = EPW // W
ETAIL = EPW - NCHUNK * W
NBUF = 6
RPT = (N // NS) // 8 * 8
RTAIL = N - NS * RPT

BLK = 2000
GRID = N // BLK


def _vmesh():
    return plsc.VectorSubcoreMesh(core_axis_name="c", subcore_axis_name="s")


NW = NC * NS


def _sc_degree(dst):
    @functools.partial(
        pl.kernel,
        out_type=jax.ShapeDtypeStruct((NW * N,), jnp.float32),
        mesh=_vmesh(),
        scratch_types=[
            pltpu.VMEM((EPW,), jnp.int32),
            pltpu.VMEM((N,), jnp.float32),
        ],
        compiler_params=pltpu.CompilerParams(needs_layout_passes=False),
    )
    def deg_kernel(dst_hbm, out_hbm, didx_all, hist):
        cid = lax.axis_index("c")
        sid = lax.axis_index("s")
        wid = cid * NS + sid
        pltpu.sync_copy(dst_hbm.at[pl.ds(wid * EPW, EPW)], didx_all)

        zeros16 = jnp.zeros((16,), jnp.float32)

        @pl.loop(0, N // 16, unroll=5)
        def _(i):
            hist[pl.ds(i * 16, 16)] = zeros16

        ones16 = jnp.full((16,), 1.0, jnp.float32)

        @pl.loop(0, EPW // 16, unroll=5)
        def _(i):
            idx = didx_all[pl.ds(i * 16, 16)]
            plsc.addupdate_scatter(hist, [idx], ones16)

        pltpu.sync_copy(hist, out_hbm.at[pl.ds(wid * N, N)])

    return deg_kernel(dst)


def _sc_aggregate(y, src, dst):
    @functools.partial(
        pl.kernel,
        out_type=jax.ShapeDtypeStruct((NC, N, D), jnp.float32),
        mesh=_vmesh(),
        scratch_types=[
            [pltpu.VMEM((W,), jnp.int32)] * NBUF,
            [pltpu.VMEM((W,), jnp.int32)] * NBUF,
            [pltpu.VMEM((W, D), jnp.float32)] * NBUF,
            pltpu.VMEM((ETAIL,), jnp.int32),
            pltpu.VMEM((ETAIL,), jnp.int32),
            pltpu.VMEM_SHARED((N, D), jnp.float32),
            [pltpu.SemaphoreType.DMA] * NBUF,
            [pltpu.SemaphoreType.DMA] * NBUF,
            [pltpu.SemaphoreType.DMA] * NBUF,
            [pltpu.SemaphoreType.DMA] * NBUF,
        ],
    )
    def agg_kernel(y_hbm, src_hbm, dst_hbm, out_hbm,
                   sidx_r, didx_r, rows_r, tsidx, tdidx, z_sh,
                   sisem, disem, gsem, ssem):
        cid = lax.axis_index("c")
        sid = lax.axis_index("s")
        r0 = sid * RPT
        pltpu.sync_copy(y_hbm.at[pl.ds(r0, RPT)], z_sh.at[pl.ds(r0, RPT)])

        @pl.when(sid == 0)
        def _():
            pltpu.sync_copy(y_hbm.at[pl.ds(NS * RPT, RTAIL)],
                            z_sh.at[pl.ds(NS * RPT, RTAIL)])

        base = (cid * NS + sid) * EPW
        plsc.subcore_barrier()

        def idx_start(q, b):
            pltpu.async_copy(src_hbm.at[pl.ds(base + q * W, W)],
                             sidx_r[b], sisem[b])
            pltpu.async_copy(dst_hbm.at[pl.ds(base + q * W, W)],
                             didx_r[b], disem[b])

        def idx_wait(q, b):
            pltpu.make_async_copy(src_hbm.at[pl.ds(base + q * W, W)],
                                  sidx_r[b], sisem[b]).wait()
            pltpu.make_async_copy(dst_hbm.at[pl.ds(base + q * W, W)],
                                  didx_r[b], disem[b]).wait()

        def gather_start(b):
            pltpu.async_copy(y_hbm.at[sidx_r[b]], rows_r[b], gsem[b])

        def gather_wait(b):
            pltpu.make_async_copy(y_hbm.at[sidx_r[b]], rows_r[b],
                                  gsem[b]).wait()

        def scatter_start(b):
            pltpu.async_copy(rows_r[b], z_sh.at[didx_r[b]], ssem[b], add=True)

        def scatter_wait(b):
            pltpu.make_async_copy(rows_r[b], z_sh.at[didx_r[b]],
                                  ssem[b]).wait()

        for b in range(NBUF):
            idx_start(b, b)

        @pl.loop(0, NCHUNK, step=NBUF)
        def _(j):
            for b in range(NBUF):
                idx_wait(j + b, b)
                gather_start(b)
            for b in range(NBUF):
                gather_wait(b)
                scatter_start(b)
            for b in range(NBUF):
                scatter_wait(b)
                pl.when(j + b + NBUF < NCHUNK)(
                    functools.partial(idx_start, j + b + NBUF, b))

        tb = base + NCHUNK * W
        pltpu.sync_copy(src_hbm.at[pl.ds(tb, ETAIL)], tsidx)
        pltpu.sync_copy(dst_hbm.at[pl.ds(tb, ETAIL)], tdidx)
        trows = rows_r[0].at[pl.ds(0, ETAIL)]
        pltpu.sync_copy(y_hbm.at[tsidx], trows)
        pltpu.sync_copy(trows, z_sh.at[tdidx], add=True)

        plsc.subcore_barrier()
        pltpu.sync_copy(z_sh.at[pl.ds(r0, RPT)], out_hbm.at[cid, pl.ds(r0, RPT)])

        @pl.when(sid == 0)
        def _():
            pltpu.sync_copy(z_sh.at[pl.ds(NS * RPT, RTAIL)],
                            out_hbm.at[cid, pl.ds(NS * RPT, RTAIL)])

    return agg_kernel(y, src, dst)


def _dis_block(degt_blk):
    deg = jnp.sum(degt_blk, axis=1, keepdims=True) + 1.0
    return lax.rsqrt(deg)


def _tc_scale_matmul(x, w, degt):
    def body(x_ref, w_ref, d_ref, o_ref):
        dis = _dis_block(d_ref[...])
        o_ref[...] = dis * jnp.dot(x_ref[...], w_ref[...],
                                   preferred_element_type=jnp.float32)

    return pl.pallas_call(
        body,
        grid=(GRID,),
        in_specs=[
            pl.BlockSpec((BLK, D), lambda i: (i, 0)),
            pl.BlockSpec((D, D), lambda i: (0, 0)),
            pl.BlockSpec((BLK, NW), lambda i: (i, 0)),
        ],
        out_specs=pl.BlockSpec((BLK, D), lambda i: (i, 0)),
        out_shape=jax.ShapeDtypeStruct((N, D), jnp.float32),
    )(x, w, degt)


def _tc_mid(z, y, degt, b, w):
    def body(z_ref, y_ref, d_ref, b_ref, w_ref, o_ref):
        dis = _dis_block(d_ref[...])
        zs = z_ref[0] + z_ref[1] - y_ref[...]
        h = jnp.maximum(dis * zs + b_ref[...], 0.0)
        o_ref[...] = dis * jnp.dot(h, w_ref[...],
                                   preferred_element_type=jnp.float32)

    return pl.pallas_call(
        body,
        grid=(GRID,),
        in_specs=[
            pl.BlockSpec((NC, BLK, D), lambda i: (0, i, 0)),
            pl.BlockSpec((BLK, D), lambda i: (i, 0)),
            pl.BlockSpec((BLK, NW), lambda i: (i, 0)),
            pl.BlockSpec((D,), lambda i: (0,)),
            pl.BlockSpec((D, D), lambda i: (0, 0)),
        ],
        out_specs=pl.BlockSpec((BLK, D), lambda i: (i, 0)),
        out_shape=jax.ShapeDtypeStruct((N, D), jnp.float32),
    )(z, y, degt, b, w)


def _tc_final(z, y, degt, b, batch2, wfc, bfc):
    dout = wfc.shape[1]

    def body(z_ref, y_ref, d_ref, b_ref, bat_ref, wfc_ref, bfc_ref, o_ref,
             acc_ref, cnt_ref):
        i = pl.program_id(0)
        dis = _dis_block(d_ref[...])
        h = jnp.maximum(dis * (z_ref[0] + z_ref[1] - y_ref[...]) + b_ref[...], 0.0)
        gids = lax.broadcasted_iota(jnp.int32, (BLK, G), 1)
        oh = (bat_ref[...] == gids).astype(jnp.float32)
        pacc = lax.dot_general(oh, h, (((0,), (0,)), ((), ())),
                               preferred_element_type=jnp.float32)
        pcnt = lax.dot_general(oh, jnp.ones((BLK, D), jnp.float32),
                               (((0,), (0,)), ((), ())),
                               preferred_element_type=jnp.float32)

        @pl.when(i == 0)
        def _():
            acc_ref[...] = jnp.zeros_like(acc_ref)
            cnt_ref[...] = jnp.zeros_like(cnt_ref)

        acc_ref[...] += pacc
        cnt_ref[...] += pcnt

        @pl.when(i == GRID - 1)
        def _():
            pooled = acc_ref[...] / jnp.maximum(cnt_ref[...], 1.0)
            o_ref[...] = (jnp.dot(pooled, wfc_ref[...],
                                  preferred_element_type=jnp.float32)
                          + bfc_ref[...])

    return pl.pallas_call(
        body,
        grid=(GRID,),
        in_specs=[
            pl.BlockSpec((NC, BLK, D), lambda i: (0, i, 0)),
            pl.BlockSpec((BLK, D), lambda i: (i, 0)),
            pl.BlockSpec((BLK, NW), lambda i: (i, 0)),
            pl.BlockSpec((D,), lambda i: (0,)),
            pl.BlockSpec((BLK, 1), lambda i: (i, 0)),
            pl.BlockSpec((D, dout), lambda i: (0, 0)),
            pl.BlockSpec((dout,), lambda i: (0,)),
        ],
        out_specs=pl.BlockSpec((G, dout), lambda i: (0, 0)),
        out_shape=jax.ShapeDtypeStruct((G, dout), jnp.float32),
        scratch_shapes=[
            pltpu.VMEM((G, D), jnp.float32),
            pltpu.VMEM((G, D), jnp.float32),
        ],
    )(z, y, degt, b, batch2, wfc, bfc)


def kernel(x, edge_index, batch, W1, b1, W2, b2, Wfc, bfc):
    src = edge_index[0].astype(jnp.int32)
    dst = edge_index[1].astype(jnp.int32)

    degp = _sc_degree(dst).reshape(NW, N)
    degt = degp.T

    y1 = _tc_scale_matmul(x, W1, degt)
    z1 = _sc_aggregate(y1, src, dst)
    y2 = _tc_mid(z1, y1, degt, b1, W2)
    z2 = _sc_aggregate(y2, src, dst)
    batch2 = batch.astype(jnp.int32).reshape(N, 1)
    return _tc_final(z2, y2, degt, b2, batch2, Wfc, bfc)
